# baseline (speedup 1.0000x reference)
import functools

import jax
import jax.numpy as jnp
from jax import lax
from jax.experimental import pallas as pl
from jax.experimental.pallas import tpu as pltpu
from jax.experimental.pallas import tpu_sc as plsc

_SEQ = 200
_BATCH = 4096
_EMBED = 64
_B = _SEQ * _BATCH

_NC, _NS = 2, 16
_NW = _NC * _NS
_G = _BATCH // _NW
_NBUF = 4
_P = 2 * _EMBED
_TP = 80

_mesh = plsc.VectorSubcoreMesh(core_axis_name="c", subcore_axis_name="s")


@functools.partial(
    pl.kernel,
    mesh=_mesh,
    out_type=jax.ShapeDtypeStruct((_B, _P), jnp.float32),
    scratch_types=[
        pltpu.VMEM((_SEQ, _G), jnp.int32),
        pltpu.VMEM((_NBUF, _G, _TP), jnp.float32),
        pltpu.SemaphoreType.DMA,
        pltpu.SemaphoreType.DMA,
        pltpu.SemaphoreType.DMA,
        pltpu.SemaphoreType.DMA,
        pltpu.SemaphoreType.DMA,
        pltpu.SemaphoreType.DMA,
        pltpu.SemaphoreType.DMA,
        pltpu.SemaphoreType.DMA,
    ],
    compiler_params=pltpu.CompilerParams(
        use_tc_tiling_on_sc=False, skip_device_barrier=True,
        disable_bounds_checks=True, disable_semaphore_checks=True),
)
def _embed_gather(idx_hbm, table_hbm, out_hbm, idx_v, praw_v,
                  g0, g1, g2, g3, w0, w1, w2, w3):
    gsem = [g0, g1, g2, g3]
    wsem = [w0, w1, w2, w3]
    wid = lax.axis_index("s") * _NC + lax.axis_index("c")
    col0 = wid * _G

    def gstart(ci, b):
        pltpu.async_copy(
            table_hbm.at[idx_v.at[ci]], praw_v.at[b], gsem[b])

    def gwait(ci, b):
        pltpu.make_async_copy(
            table_hbm.at[idx_v.at[ci]], praw_v.at[b], gsem[b]).wait()

    def wb(ci, b, sem):
        return pltpu.make_async_copy(
            praw_v.at[b, pl.ds(0, _G), pl.ds(0, _EMBED)],
            out_hbm.at[pl.ds(ci * _BATCH + col0, _G), pl.ds(0, _EMBED)],
            sem)

    pltpu.sync_copy(idx_hbm.at[:, pl.ds(col0, _G)], idx_v)

    gstart(0, 0)
    gstart(1, 1)

    def group(g, carry):
        ci0 = g * _NBUF
        for q in range(_NBUF):
            ci = ci0 + q
            gwait(ci, q)

            @pl.when(ci >= 2)
            def _():
                wb(ci - 2, (q + 2) % _NBUF, wsem[(q + 2) % _NBUF]).wait()

            @pl.when(ci + 2 < _SEQ)
            def _():
                gstart(ci + 2, (q + 2) % _NBUF)

            wb(ci, q, wsem[q]).start()
        return carry

    lax.fori_loop(0, _SEQ // _NBUF, group, 0)

    for ci in range(_SEQ - 2, _SEQ):
        wb(ci, ci % _NBUF, wsem[ci % _NBUF]).wait()


def kernel(indices, table):
    tpad = jnp.pad(table, ((0, 0), (0, _TP - _EMBED)))
    outp = _embed_gather(indices.astype(jnp.int32), tpad)
    return outp[:, :_EMBED].reshape(_SEQ, _BATCH, _EMBED)


# speedup vs baseline: 1.5473x; 1.5473x over previous
import functools

import jax
import jax.numpy as jnp
from jax import lax
from jax.experimental import pallas as pl
from jax.experimental.pallas import tpu as pltpu
from jax.experimental.pallas import tpu_sc as plsc

_SEQ = 200
_BATCH = 4096
_EMBED = 64
_B = _SEQ * _BATCH

_NC, _NS = 2, 16
_NW = _NC * _NS
_G = _BATCH // _NW
_NBUF = 4
_P = 2 * _EMBED

_mesh = plsc.VectorSubcoreMesh(core_axis_name="c", subcore_axis_name="s")


@functools.partial(
    pl.kernel,
    mesh=_mesh,
    out_type=jax.ShapeDtypeStruct((_B, _P), jnp.float32),
    scratch_types=[
        pltpu.VMEM((_SEQ, _G), jnp.int32),
        pltpu.VMEM((_NBUF, _G, _P), jnp.float32),
        pltpu.SemaphoreType.DMA,
        pltpu.SemaphoreType.DMA,
        pltpu.SemaphoreType.DMA,
        pltpu.SemaphoreType.DMA,
        pltpu.SemaphoreType.DMA,
        pltpu.SemaphoreType.DMA,
        pltpu.SemaphoreType.DMA,
        pltpu.SemaphoreType.DMA,
    ],
    compiler_params=pltpu.CompilerParams(
        use_tc_tiling_on_sc=False, skip_device_barrier=True,
        disable_bounds_checks=True, disable_semaphore_checks=True),
)
def _embed_gather(idx_hbm, table_hbm, out_hbm, idx_v, praw_v,
                  g0, g1, g2, g3, w0, w1, w2, w3):
    gsem = [g0, g1, g2, g3]
    wsem = [w0, w1, w2, w3]
    wid = lax.axis_index("s") * _NC + lax.axis_index("c")
    col0 = wid * _G

    def gstart(ci, b):
        pltpu.async_copy(
            table_hbm.at[idx_v.at[ci]], praw_v.at[b], gsem[b])

    def gwait(ci, b):
        pltpu.make_async_copy(
            table_hbm.at[idx_v.at[ci]], praw_v.at[b], gsem[b]).wait()

    def wb(ci, b, sem):
        return pltpu.make_async_copy(
            praw_v.at[b, pl.ds(0, _G), pl.ds(0, _EMBED)],
            out_hbm.at[pl.ds(ci * _BATCH + col0, _G), pl.ds(0, _EMBED)],
            sem)

    pltpu.sync_copy(idx_hbm.at[:, pl.ds(col0, _G)], idx_v)

    gstart(0, 0)
    gstart(1, 1)

    def group(g, carry):
        ci0 = g * _NBUF
        for q in range(_NBUF):
            ci = ci0 + q
            gwait(ci, q)

            @pl.when(ci >= 2)
            def _():
                wb(ci - 2, (q + 2) % _NBUF, wsem[(q + 2) % _NBUF]).wait()

            @pl.when(ci + 2 < _SEQ)
            def _():
                gstart(ci + 2, (q + 2) % _NBUF)

            wb(ci, q, wsem[q]).start()
        return carry

    lax.fori_loop(0, _SEQ // _NBUF, group, 0)

    for ci in range(_SEQ - 2, _SEQ):
        wb(ci, ci % _NBUF, wsem[ci % _NBUF]).wait()


def kernel(indices, table):
    tpad = jnp.pad(table, ((0, 0), (0, _P - _EMBED)))
    outp = _embed_gather(indices.astype(jnp.int32), tpad)
    return outp[:, :_EMBED].reshape(_SEQ, _BATCH, _EMBED)


# speedup vs baseline: 1.5569x; 1.0062x over previous
import functools

import jax
import jax.numpy as jnp
from jax import lax
from jax.experimental import pallas as pl
from jax.experimental.pallas import tpu as pltpu
from jax.experimental.pallas import tpu_sc as plsc

_SEQ = 200
_BATCH = 4096
_EMBED = 64
_B = _SEQ * _BATCH

_NC, _NS = 2, 16
_NW = _NC * _NS
_G = _BATCH // _NW
_NBUF = 4
_P = 2 * _EMBED

_mesh = plsc.VectorSubcoreMesh(core_axis_name="c", subcore_axis_name="s")


@functools.partial(
    pl.kernel,
    mesh=_mesh,
    out_type=jax.ShapeDtypeStruct((_B, _P), jnp.float32),
    scratch_types=[
        pltpu.VMEM((_SEQ, _G), jnp.int32),
        pltpu.VMEM((_NBUF, _G, _EMBED), jnp.float32),
        pltpu.SemaphoreType.DMA,
        pltpu.SemaphoreType.DMA,
        pltpu.SemaphoreType.DMA,
        pltpu.SemaphoreType.DMA,
        pltpu.SemaphoreType.DMA,
        pltpu.SemaphoreType.DMA,
        pltpu.SemaphoreType.DMA,
        pltpu.SemaphoreType.DMA,
    ],
    compiler_params=pltpu.CompilerParams(
        use_tc_tiling_on_sc=False, skip_device_barrier=True,
        disable_bounds_checks=True, disable_semaphore_checks=True),
)
def _embed_gather(idx_hbm, table_hbm, out_hbm, idx_v, praw_v,
                  g0, g1, g2, g3, w0, w1, w2, w3):
    gsem = [g0, g1, g2, g3]
    wsem = [w0, w1, w2, w3]
    wid = lax.axis_index("s") * _NC + lax.axis_index("c")
    col0 = wid * _G

    def gstart(ci, b):
        pltpu.async_copy(
            table_hbm.at[idx_v.at[ci]], praw_v.at[b], gsem[b])

    def gwait(ci, b):
        pltpu.make_async_copy(
            table_hbm.at[idx_v.at[ci]], praw_v.at[b], gsem[b]).wait()

    def wb(ci, b, sem):
        return pltpu.make_async_copy(
            praw_v.at[b],
            out_hbm.at[pl.ds(ci * _BATCH + col0, _G), pl.ds(0, _EMBED)],
            sem)

    pltpu.sync_copy(idx_hbm.at[:, pl.ds(col0, _G)], idx_v)

    gstart(0, 0)
    gstart(1, 1)

    def group(g, carry):
        ci0 = g * _NBUF
        for q in range(_NBUF):
            ci = ci0 + q
            gwait(ci, q)

            @pl.when(ci >= 2)
            def _():
                wb(ci - 2, (q + 2) % _NBUF, wsem[(q + 2) % _NBUF]).wait()

            @pl.when(ci + 2 < _SEQ)
            def _():
                gstart(ci + 2, (q + 2) % _NBUF)

            wb(ci, q, wsem[q]).start()
        return carry

    lax.fori_loop(0, _SEQ // _NBUF, group, 0)

    for ci in range(_SEQ - 2, _SEQ):
        wb(ci, ci % _NBUF, wsem[ci % _NBUF]).wait()


def kernel(indices, table):
    outp = _embed_gather(indices.astype(jnp.int32), table)
    return outp[:, :_EMBED].reshape(_SEQ, _BATCH, _EMBED)
